# baseline (speedup 1.0000x reference)
import jax
import jax.numpy as jnp
from jax import lax
from jax.experimental import pallas as pl
from jax.experimental.pallas import tpu as pltpu
from jax.experimental.pallas import tpu_sc as plsc

N = 10000
E = 160000
D = 256
QD = D // 4
NS = 16
EPT = E // NS
CK = 80
NCH = EPT // CK
N_PAD = 10112
SLAB = N_PAD // NS
ZR = 8


def _prep_body(h_ref, att_ref, ab_ref, h4_ref):
    hh = h_ref[...]
    ab_ref[:, pl.ds(0, N)] = lax.dot_general(
        att_ref[...], hh,
        dimension_numbers=(((1,), (1,)), ((), ())),
        preferred_element_type=jnp.float32,
    )
    for q in range(4):
        h4_ref[q, pl.ds(0, N), :] = hh[:, q * QD:(q + 1) * QD]


def _sc_body(h4, row3, col3, adj3, ab, outp,
             row2d, col2d, adj2d, w2d, av, bv, gbuf, zbuf, statv,
             acc, stat_sh, gs0, gs1, ss0, ss1):
    c = lax.axis_index("c")
    s = lax.axis_index("s")

    pltpu.sync_copy(row3.at[s], row2d)
    pltpu.sync_copy(col3.at[s], col2d)
    pltpu.sync_copy(adj3.at[s], adj2d)
    pltpu.sync_copy(ab.at[0], av)
    pltpu.sync_copy(ab.at[1], bv)

    z16 = jnp.zeros((16,), jnp.float32)
    def _z_body(i, carry):
        for q in range(QD // 16):
            zbuf[i, pl.ds(q * 16, 16)] = z16
        return carry
    lax.fori_loop(0, ZR, _z_body, 0)

    def _p1_body(j, m):
        for k in range(CK // 16):
            sl = pl.ds(k * 16, 16)
            r16 = row2d[j, sl]
            c16 = col2d[j, sl]
            a16 = plsc.load_gather(av, [r16])
            b16 = plsc.load_gather(bv, [c16])
            s16 = a16 + b16
            s16 = jnp.where(s16 >= 0.0, s16, 0.2 * s16)
            w2d[j, sl] = s16
            m = jnp.maximum(m, s16)
        return m
    m = lax.fori_loop(0, NCH, _p1_body,
                      jnp.full((16,), -3.0e38, jnp.float32))

    statv[0, :] = m
    pltpu.sync_copy(statv.at[0], stat_sh.at[0, s])
    plsc.subcore_barrier()
    pltpu.sync_copy(stat_sh.at[0], statv)
    gmv = statv[0, :]
    for k in range(1, NS):
        gmv = jnp.maximum(gmv, statv[k, :])
    gm = jnp.max(gmv)

    def _p2_body(j, sv):
        for k in range(CK // 16):
            sl = pl.ds(k * 16, 16)
            e16 = jnp.exp(w2d[j, sl] - gm)
            sv = sv + e16
            w2d[j, sl] = e16 * adj2d[j, sl]
        return sv
    sv = lax.fori_loop(0, NCH, _p2_body, jnp.zeros((16,), jnp.float32))
    statv[0, :] = sv
    pltpu.sync_copy(statv.at[0], stat_sh.at[1, s])
    plsc.subcore_barrier()
    pltpu.sync_copy(stat_sh.at[1], statv)
    zv = statv[0, :]
    for k in range(1, NS):
        zv = zv + statv[k, :]
    inv_z = 1.0 / (jnp.zeros((16,), jnp.float32) + jnp.sum(zv))

    def _wz_body(j, carry):
        for k in range(CK // 16):
            sl = pl.ds(k * 16, 16)
            w2d[j, sl] = w2d[j, sl] * inv_z
        return carry
    lax.fori_loop(0, NCH, _wz_body, 0)

    b0 = gbuf.at[0]
    b1 = gbuf.at[1]
    for p in range(2):
        qo = pl.multiple_of((2 * c + p) * QD, QD)
        hq = h4.at[2 * c + p]

        def _zc_body(t, carry):
            off = pl.multiple_of(s * SLAB + t * ZR, 8)
            pltpu.async_copy(zbuf, acc.at[pl.ds(off, ZR)], gs0)
            return carry
        lax.fori_loop(0, SLAB // ZR, _zc_body, 0)

        def _zw_body(t, carry):
            off = pl.multiple_of(s * SLAB + t * ZR, 8)
            pltpu.make_async_copy(zbuf, acc.at[pl.ds(off, ZR)], gs0).wait()
            return carry
        lax.fori_loop(0, SLAB // ZR, _zw_body, 0)
        plsc.subcore_barrier()

        def _scale(j, buf):
            jsplat = jnp.zeros((16,), jnp.int32) + j

            def _sc(eb, c2):
                e0 = eb * 2
                for u in range(2):
                    esplat = jnp.zeros((16,), jnp.int32) + (e0 + u)
                    wv = plsc.load_gather(w2d, [jsplat, esplat])
                    for q in range(QD // 16):
                        sl = pl.ds(q * 16, 16)
                        buf[e0 + u, sl] = buf[e0 + u, sl] * wv
                return c2
            lax.fori_loop(0, CK // 2, _sc, 0)

        pltpu.async_copy(hq.at[col2d.at[0]], b0, gs0)
        pltpu.async_copy(hq.at[col2d.at[1]], b1, gs1)

        def _pair(j2, carry):
            j0 = 2 * j2
            j1 = j0 + 1
            pltpu.make_async_copy(hq.at[col2d.at[j0]], b0, gs0).wait()
            _scale(j0, b0)
            pltpu.async_copy(b0, acc.at[row2d.at[j0]], ss0, add=True)
            pltpu.make_async_copy(hq.at[col2d.at[j1]], b1, gs1).wait()
            _scale(j1, b1)
            pltpu.async_copy(b1, acc.at[row2d.at[j1]], ss1, add=True)
            pltpu.make_async_copy(b0, acc.at[row2d.at[j0]], ss0).wait()
            pltpu.async_copy(hq.at[col2d.at[j0 + 2]], b0, gs0)
            pltpu.make_async_copy(b1, acc.at[row2d.at[j1]], ss1).wait()

            @pl.when(j2 < (NCH - 3) // 2)
            def _():
                pltpu.async_copy(hq.at[col2d.at[j1 + 2]], b1, gs1)
            return carry
        lax.fori_loop(0, (NCH - 1) // 2, _pair, 0)

        jl = NCH - 1
        pltpu.make_async_copy(hq.at[col2d.at[jl]], b0, gs0).wait()
        _scale(jl, b0)
        pltpu.async_copy(b0, acc.at[row2d.at[jl]], ss0, add=True)
        pltpu.make_async_copy(b0, acc.at[row2d.at[jl]], ss0).wait()

        plsc.subcore_barrier()
        slab_off = pl.multiple_of(s * SLAB, 8)
        pltpu.sync_copy(acc.at[pl.ds(slab_off, SLAB)],
                        outp.at[pl.ds(slab_off, SLAB), pl.ds(qo, QD)])


_sc_call = pl.kernel(
    _sc_body,
    mesh=plsc.VectorSubcoreMesh(core_axis_name="c", subcore_axis_name="s"),
    compiler_params=pltpu.CompilerParams(
        needs_layout_passes=False, use_tc_tiling_on_sc=False),
    out_type=jax.ShapeDtypeStruct((N_PAD, D), jnp.float32),
    scratch_types=[
        pltpu.VMEM((NCH, CK), jnp.int32),
        pltpu.VMEM((NCH, CK), jnp.int32),
        pltpu.VMEM((NCH, CK), jnp.float32),
        pltpu.VMEM((NCH, CK), jnp.float32),
        pltpu.VMEM((N_PAD,), jnp.float32),
        pltpu.VMEM((N_PAD,), jnp.float32),
        pltpu.VMEM((2, CK, QD), jnp.float32),
        pltpu.VMEM((ZR, QD), jnp.float32),
        pltpu.VMEM((NS, 16), jnp.float32),
        pltpu.VMEM_SHARED((N_PAD, QD), jnp.float32),
        pltpu.VMEM_SHARED((2, NS, 16), jnp.float32),
        pltpu.SemaphoreType.DMA,
        pltpu.SemaphoreType.DMA,
        pltpu.SemaphoreType.DMA,
        pltpu.SemaphoreType.DMA,
    ],
)


def kernel(h, edge_index, adj_values, att):
    h = h.astype(jnp.float32)
    row = edge_index[0].astype(jnp.int32)
    col = edge_index[1].astype(jnp.int32)
    att2 = att.astype(jnp.float32).reshape(2, D)

    ab, h4 = pl.pallas_call(
        _prep_body,
        out_shape=(
            jax.ShapeDtypeStruct((2, N_PAD), jnp.float32),
            jax.ShapeDtypeStruct((4, N_PAD, QD), jnp.float32),
        ),
    )(h, att2)

    row3 = row.reshape(NS, NCH, CK)
    col3 = col.reshape(NS, NCH, CK)
    adj3 = adj_values.astype(jnp.float32).reshape(NS, NCH, CK)

    outp = _sc_call(h4, row3, col3, adj3, ab)
    return outp[:N]


# speedup vs baseline: 1.0056x; 1.0056x over previous
import jax
import jax.numpy as jnp
from jax import lax
from jax.experimental import pallas as pl
from jax.experimental.pallas import tpu as pltpu
from jax.experimental.pallas import tpu_sc as plsc

N = 10000
E = 160000
D = 256
QD = D // 4
NS = 16
EPT = E // NS
CK = 80
NCH = EPT // CK
N_PAD = 10112
SLAB = N_PAD // NS
ZR = 8


def _prep_body(h_ref, att_ref, ab_ref, h4_ref):
    hh = h_ref[...]
    ab_ref[:, pl.ds(0, N)] = lax.dot_general(
        att_ref[...], hh,
        dimension_numbers=(((1,), (1,)), ((), ())),
        preferred_element_type=jnp.float32,
    )
    for q in range(4):
        h4_ref[q, pl.ds(0, N), :] = hh[:, q * QD:(q + 1) * QD]


def _sc_body(h4, row3, col3, adj3, ab, outp,
             row2d, col2d, adj2d, w2d, av, bv, gbuf, zbuf, statv,
             acc, stat_sh, gs0, gs1, ss0, ss1):
    c = lax.axis_index("c")
    s = lax.axis_index("s")

    pltpu.sync_copy(row3.at[s], row2d)
    pltpu.sync_copy(col3.at[s], col2d)
    pltpu.sync_copy(adj3.at[s], adj2d)
    pltpu.sync_copy(ab.at[0], av)
    pltpu.sync_copy(ab.at[1], bv)

    z16 = jnp.zeros((16,), jnp.float32)
    def _z_body(i, carry):
        for q in range(QD // 16):
            zbuf[i, pl.ds(q * 16, 16)] = z16
        return carry
    lax.fori_loop(0, ZR, _z_body, 0)

    def _p1_body(j, m):
        for k in range(CK // 16):
            sl = pl.ds(k * 16, 16)
            r16 = row2d[j, sl]
            c16 = col2d[j, sl]
            a16 = plsc.load_gather(av, [r16])
            b16 = plsc.load_gather(bv, [c16])
            s16 = a16 + b16
            s16 = jnp.where(s16 >= 0.0, s16, 0.2 * s16)
            w2d[j, sl] = s16
            m = jnp.maximum(m, s16)
        return m
    m = lax.fori_loop(0, NCH, _p1_body,
                      jnp.full((16,), -3.0e38, jnp.float32))

    statv[0, :] = m
    pltpu.sync_copy(statv.at[0], stat_sh.at[0, s])
    plsc.subcore_barrier()
    pltpu.sync_copy(stat_sh.at[0], statv)
    gmv = statv[0, :]
    for k in range(1, NS):
        gmv = jnp.maximum(gmv, statv[k, :])
    gm = jnp.max(gmv)

    def _p2_body(j, sv):
        for k in range(CK // 16):
            sl = pl.ds(k * 16, 16)
            e16 = jnp.exp(w2d[j, sl] - gm)
            sv = sv + e16
            w2d[j, sl] = e16 * adj2d[j, sl]
        return sv
    sv = lax.fori_loop(0, NCH, _p2_body, jnp.zeros((16,), jnp.float32))
    statv[0, :] = sv
    pltpu.sync_copy(statv.at[0], stat_sh.at[1, s])
    plsc.subcore_barrier()
    pltpu.sync_copy(stat_sh.at[1], statv)
    zv = statv[0, :]
    for k in range(1, NS):
        zv = zv + statv[k, :]
    inv_z = 1.0 / (jnp.zeros((16,), jnp.float32) + jnp.sum(zv))

    def _wz_body(j, carry):
        for k in range(CK // 16):
            sl = pl.ds(k * 16, 16)
            w2d[j, sl] = w2d[j, sl] * inv_z
        return carry
    lax.fori_loop(0, NCH, _wz_body, 0)

    b0 = gbuf.at[0]
    b1 = gbuf.at[1]
    for p in range(2):
        qo = pl.multiple_of((2 * c + p) * QD, QD)
        hq = h4.at[2 * c + p]

        def _zc_body(t, carry):
            off = pl.multiple_of(s * SLAB + t * ZR, 8)
            pltpu.async_copy(zbuf, acc.at[pl.ds(off, ZR)], gs0)
            return carry
        lax.fori_loop(0, SLAB // ZR, _zc_body, 0)

        def _zw_body(t, carry):
            off = pl.multiple_of(s * SLAB + t * ZR, 8)
            pltpu.make_async_copy(zbuf, acc.at[pl.ds(off, ZR)], gs0).wait()
            return carry
        lax.fori_loop(0, SLAB // ZR, _zw_body, 0)
        plsc.subcore_barrier()

        def _scale(j, buf):
            jsplat = jnp.zeros((16,), jnp.int32) + j

            def _sc(eb, c2):
                e0 = eb * 4
                for u in range(4):
                    esplat = jnp.zeros((16,), jnp.int32) + (e0 + u)
                    wv = plsc.load_gather(w2d, [jsplat, esplat])
                    for q in range(QD // 16):
                        sl = pl.ds(q * 16, 16)
                        buf[e0 + u, sl] = buf[e0 + u, sl] * wv
                return c2
            lax.fori_loop(0, CK // 4, _sc, 0)

        pltpu.async_copy(hq.at[col2d.at[0]], b0, gs0)
        pltpu.async_copy(hq.at[col2d.at[1]], b1, gs1)

        def _pair(j2, carry):
            j0 = 2 * j2
            j1 = j0 + 1
            pltpu.make_async_copy(hq.at[col2d.at[j0]], b0, gs0).wait()
            _scale(j0, b0)
            pltpu.async_copy(b0, acc.at[row2d.at[j0]], ss0, add=True)
            pltpu.make_async_copy(hq.at[col2d.at[j1]], b1, gs1).wait()
            _scale(j1, b1)
            pltpu.async_copy(b1, acc.at[row2d.at[j1]], ss1, add=True)
            pltpu.make_async_copy(b0, acc.at[row2d.at[j0]], ss0).wait()
            pltpu.async_copy(hq.at[col2d.at[j0 + 2]], b0, gs0)
            pltpu.make_async_copy(b1, acc.at[row2d.at[j1]], ss1).wait()

            @pl.when(j2 < (NCH - 3) // 2)
            def _():
                pltpu.async_copy(hq.at[col2d.at[j1 + 2]], b1, gs1)
            return carry
        lax.fori_loop(0, (NCH - 1) // 2, _pair, 0)

        jl = NCH - 1
        pltpu.make_async_copy(hq.at[col2d.at[jl]], b0, gs0).wait()
        _scale(jl, b0)
        pltpu.async_copy(b0, acc.at[row2d.at[jl]], ss0, add=True)
        pltpu.make_async_copy(b0, acc.at[row2d.at[jl]], ss0).wait()

        plsc.subcore_barrier()
        slab_off = pl.multiple_of(s * SLAB, 8)
        pltpu.sync_copy(acc.at[pl.ds(slab_off, SLAB)],
                        outp.at[pl.ds(slab_off, SLAB), pl.ds(qo, QD)])


_sc_call = pl.kernel(
    _sc_body,
    mesh=plsc.VectorSubcoreMesh(core_axis_name="c", subcore_axis_name="s"),
    compiler_params=pltpu.CompilerParams(
        needs_layout_passes=False, use_tc_tiling_on_sc=False),
    out_type=jax.ShapeDtypeStruct((N_PAD, D), jnp.float32),
    scratch_types=[
        pltpu.VMEM((NCH, CK), jnp.int32),
        pltpu.VMEM((NCH, CK), jnp.int32),
        pltpu.VMEM((NCH, CK), jnp.float32),
        pltpu.VMEM((NCH, CK), jnp.float32),
        pltpu.VMEM((N_PAD,), jnp.float32),
        pltpu.VMEM((N_PAD,), jnp.float32),
        pltpu.VMEM((2, CK, QD), jnp.float32),
        pltpu.VMEM((ZR, QD), jnp.float32),
        pltpu.VMEM((NS, 16), jnp.float32),
        pltpu.VMEM_SHARED((N_PAD, QD), jnp.float32),
        pltpu.VMEM_SHARED((2, NS, 16), jnp.float32),
        pltpu.SemaphoreType.DMA,
        pltpu.SemaphoreType.DMA,
        pltpu.SemaphoreType.DMA,
        pltpu.SemaphoreType.DMA,
    ],
)


def kernel(h, edge_index, adj_values, att):
    h = h.astype(jnp.float32)
    row = edge_index[0].astype(jnp.int32)
    col = edge_index[1].astype(jnp.int32)
    att2 = att.astype(jnp.float32).reshape(2, D)

    ab, h4 = pl.pallas_call(
        _prep_body,
        out_shape=(
            jax.ShapeDtypeStruct((2, N_PAD), jnp.float32),
            jax.ShapeDtypeStruct((4, N_PAD, QD), jnp.float32),
        ),
    )(h, att2)

    row3 = row.reshape(NS, NCH, CK)
    col3 = col.reshape(NS, NCH, CK)
    adj3 = adj_values.astype(jnp.float32).reshape(NS, NCH, CK)

    outp = _sc_call(h4, row3, col3, adj3, ab)
    return outp[:N]
